# Initial kernel scaffold; baseline (speedup 1.0000x reference)
import jax
import jax.numpy as jnp
from jax.experimental import pallas as pl


def kernel(x, W_out, b_out, W_bid, b_bid):
    raise NotImplementedError("write your pallas kernel here")


# speedup vs baseline: 4.3274x; 4.3274x over previous
import jax
import jax.numpy as jnp
from jax.experimental import pallas as pl
from jax.experimental.pallas import tpu as pltpu

B = 2048
D = 768
O = 768
E = 16
NEG_INF = float("-inf")


def _fused_kernel(x_ref, w_out_ref, b_out_ref, w_bid_ref, b_bid_ref,
                  final_ref, idx_ref, bids_ref, xb_ref, coeff_ref):
    e = pl.program_id(0)
    iota = jax.lax.broadcasted_iota(jnp.int32, (B, E), 1)

    @pl.when(e == 0)
    def _init():
        x = x_ref[...]
        bids = jax.lax.dot_general(
            x, w_bid_ref[...], (((1,), (1,)), ((), ())),
            preferred_element_type=jnp.float32) + b_bid_ref[...]
        bids_ref[...] = bids
        max0 = jnp.max(bids, axis=1, keepdims=True)
        i0 = jnp.min(jnp.where(bids == max0, iota, E), axis=1, keepdims=True)
        masked = jnp.where(iota == i0, NEG_INF, bids)
        max1 = jnp.max(masked, axis=1, keepdims=True)
        i1 = jnp.min(jnp.where(masked == max1, iota, E), axis=1, keepdims=True)
        idx_ref[...] = jnp.concatenate([i0, i1], axis=1)
        coeff = jnp.where((iota == i0) | (iota == i1), 0.5, 0.0)
        coeff_ref[...] = coeff
        final_ref[...] = jnp.dot(coeff, b_out_ref[...],
                                 preferred_element_type=jnp.float32)
        xb_ref[...] = x.astype(jnp.bfloat16)

    c_e = jnp.sum(jnp.where(iota == e, coeff_ref[...], 0.0),
                  axis=1, keepdims=True)
    prod = jnp.dot(xb_ref[...], w_out_ref[0].astype(jnp.bfloat16),
                   preferred_element_type=jnp.float32)
    final_ref[...] += c_e * prod


def kernel(x, W_out, b_out, W_bid, b_bid):
    final, idx, bids = pl.pallas_call(
        _fused_kernel,
        grid=(E,),
        in_specs=[
            pl.BlockSpec((B, D), lambda e: (0, 0)),
            pl.BlockSpec((1, D, O), lambda e: (e, 0, 0)),
            pl.BlockSpec((E, O), lambda e: (0, 0)),
            pl.BlockSpec((E, D), lambda e: (0, 0)),
            pl.BlockSpec((1, E), lambda e: (0, 0)),
        ],
        out_specs=[
            pl.BlockSpec((B, O), lambda e: (0, 0)),
            pl.BlockSpec((B, 2), lambda e: (0, 0)),
            pl.BlockSpec((B, E), lambda e: (0, 0)),
        ],
        out_shape=[
            jax.ShapeDtypeStruct((B, O), jnp.float32),
            jax.ShapeDtypeStruct((B, 2), jnp.int32),
            jax.ShapeDtypeStruct((B, E), jnp.float32),
        ],
        scratch_shapes=[
            pltpu.VMEM((B, D), jnp.bfloat16),
            pltpu.VMEM((B, E), jnp.float32),
        ],
    )(x, W_out, b_out, W_bid, b_bid.reshape(1, E))
    return final, idx, bids
